# baseline (speedup 1.0000x reference)
import functools

import jax
import jax.numpy as jnp
from jax import lax
from jax.experimental import pallas as pl
from jax.experimental.pallas import tpu as pltpu
from jax.experimental.pallas import tpu_sc as plsc

NEXP = 16
NF = 12
TOPK = 2
D = 1024
F = 2048
T = 4096
B = 256
NB = (T * TOPK) // B + NF
NPAD = NB * B
ZROW = NPAD - 1
RB = 512
NW = 32

_SC_MESH = plsc.VectorSubcoreMesh(core_axis_name="c", subcore_axis_name="s")


def _router_body(x_ref, wg_ref, cwf_ref, cconst_ref,
                 cheap_ref, gates_ref, idx_ref):
    xb = x_ref[...]
    logits = jnp.dot(xb, wg_ref[...], preferred_element_type=jnp.float32)
    m = jnp.max(logits, axis=1, keepdims=True)
    ex = jnp.exp(logits - m)
    p = ex / jnp.sum(ex, axis=1, keepdims=True)
    iota = lax.broadcasted_iota(jnp.int32, (RB, NEXP), 1)
    g1 = jnp.max(p, axis=1, keepdims=True)
    i1 = jnp.min(jnp.where(p == g1, iota, NEXP), axis=1, keepdims=True)
    p2 = jnp.where(iota == i1, -jnp.inf, p)
    g2 = jnp.max(p2, axis=1, keepdims=True)
    i2 = jnp.min(jnp.where(p2 == g2, iota, NEXP), axis=1, keepdims=True)
    g1z = jnp.where(i1 == NEXP - 1, 0.0, g1)
    g2z = jnp.where(i2 == NEXP - 1, 0.0, g2)
    s = g1z + g2z
    gn1 = g1z / s
    gn2 = g2z / s
    t2 = xb * 2.0
    cl = jnp.dot(t2, cwf_ref[...], preferred_element_type=jnp.float32)
    cc = cconst_ref[...]
    cheap = jnp.zeros_like(xb)
    for j in range(2):
        lj = cl[:, 2 * j:2 * j + 2]
        mj = jnp.max(lj, axis=1, keepdims=True)
        ej = jnp.exp(lj - mj)
        wj = ej / jnp.sum(ej, axis=1, keepdims=True)
        ge = (jnp.where(i1 == NF + j, gn1, 0.0)
              + jnp.where(i2 == NF + j, gn2, 0.0))
        cheap = cheap + ge * (wj[:, 0:1] * t2 + wj[:, 1:2] * cc[j:j + 1, :])
    ge_c = (jnp.where(i1 == NEXP - 2, gn1, 0.0)
            + jnp.where(i2 == NEXP - 2, gn2, 0.0))
    cheap = cheap + ge_c * t2
    cheap_ref[...] = cheap
    gates_ref[...] = jnp.concatenate([gn1, gn2], axis=1)
    idx_ref[...] = jnp.concatenate([i1, i2], axis=1).astype(jnp.int32)


def _router(xf, wg, cwf, cconst):
    return pl.pallas_call(
        _router_body,
        grid=(T // RB,),
        in_specs=[
            pl.BlockSpec((RB, D), lambda i: (i, 0)),
            pl.BlockSpec((D, NEXP), lambda i: (0, 0)),
            pl.BlockSpec((D, 4), lambda i: (0, 0)),
            pl.BlockSpec((2, D), lambda i: (0, 0)),
        ],
        out_specs=[
            pl.BlockSpec((RB, D), lambda i: (i, 0)),
            pl.BlockSpec((RB, TOPK), lambda i: (i, 0)),
            pl.BlockSpec((RB, TOPK), lambda i: (i, 0)),
        ],
        out_shape=[
            jax.ShapeDtypeStruct((T, D), jnp.float32),
            jax.ShapeDtypeStruct((T, TOPK), jnp.float32),
            jax.ShapeDtypeStruct((T, TOPK), jnp.int32),
        ],
    )(xf, wg, cwf, cconst)


GC = 32
GPW = NPAD // NW
GCH = GPW // GC


@functools.partial(
    pl.kernel,
    mesh=_SC_MESH,
    out_type=jax.ShapeDtypeStruct((NPAD, D), jnp.float32),
    scratch_types=[
        pltpu.VMEM((GC,), jnp.int32),
        pltpu.VMEM((GC, D), jnp.float32),
        pltpu.SemaphoreType.DMA,
    ],
)
def _sc_gather(xf_hbm, tok_hbm, xs_hbm, idx_v, rows_v, sem):
    wid = lax.axis_index("s") * 2 + lax.axis_index("c")
    base = wid * GPW
    for c in range(GCH):
        off = base + c * GC
        pltpu.sync_copy(tok_hbm.at[pl.ds(off, GC)], idx_v)
        pltpu.async_copy(xf_hbm.at[idx_v], rows_v, sem).wait()
        pltpu.sync_copy(rows_v, xs_hbm.at[pl.ds(off, GC)])


def _ffn_body(be_ref, na_ref, xs_ref, w1_ref, b1_ref, w2_ref, b2_ref, g_ref,
              ys_ref):
    b = pl.program_id(0)

    @pl.when(b < na_ref[0])
    def _compute():
        xb = (xs_ref[...] * 2.0).astype(jnp.bfloat16)
        h = jnp.dot(xb, w1_ref[0], preferred_element_type=jnp.float32)
        h = jnp.maximum(h + b1_ref[0], 0.0).astype(jnp.bfloat16)
        y = jnp.dot(h, w2_ref[0], preferred_element_type=jnp.float32)
        ys_ref[...] = (y + b2_ref[0]) * g_ref[...]

    @pl.when(b >= na_ref[0])
    def _zero():
        ys_ref[...] = jnp.zeros_like(ys_ref)


def _ffn(block_expert, n_active, xs, W1b, b1r, W2b, b2r, gate_col):
    grid_spec = pltpu.PrefetchScalarGridSpec(
        num_scalar_prefetch=2,
        grid=(NB,),
        in_specs=[
            pl.BlockSpec((B, D), lambda b, be, na: (b, 0)),
            pl.BlockSpec((1, D, F), lambda b, be, na: (be[b], 0, 0)),
            pl.BlockSpec((1, 1, F), lambda b, be, na: (be[b], 0, 0)),
            pl.BlockSpec((1, F, D), lambda b, be, na: (be[b], 0, 0)),
            pl.BlockSpec((1, 1, D), lambda b, be, na: (be[b], 0, 0)),
            pl.BlockSpec((B, 1), lambda b, be, na: (b, 0)),
        ],
        out_specs=pl.BlockSpec((B, D), lambda b, be, na: (b, 0)),
    )
    return pl.pallas_call(
        _ffn_body,
        grid_spec=grid_spec,
        out_shape=jax.ShapeDtypeStruct((NPAD, D), jnp.float32),
        compiler_params=pltpu.CompilerParams(
            dimension_semantics=("arbitrary",)),
    )(block_expert, n_active, xs, W1b, b1r, W2b, b2r, gate_col)


CC = 16
TPW = T // NW
CCH = TPW // CC


@functools.partial(
    pl.kernel,
    mesh=_SC_MESH,
    out_type=jax.ShapeDtypeStruct((T, D), jnp.float32),
    scratch_types=[
        pltpu.VMEM((CC,), jnp.int32),
        pltpu.VMEM((CC,), jnp.int32),
        pltpu.VMEM((CC, D), jnp.float32),
        pltpu.VMEM((CC, D), jnp.float32),
        pltpu.VMEM((CC, D), jnp.float32),
        pltpu.SemaphoreType.DMA,
        pltpu.SemaphoreType.DMA,
    ],
)
def _sc_combine(ys_hbm, cheap_hbm, pos0_hbm, pos1_hbm, out_hbm,
                idx0_v, idx1_v, r0_v, r1_v, acc_v, sem, sem2):
    wid = lax.axis_index("s") * 2 + lax.axis_index("c")
    base = wid * TPW
    for c in range(CCH):
        off = base + c * CC
        pltpu.sync_copy(pos0_hbm.at[pl.ds(off, CC)], idx0_v)
        pltpu.sync_copy(pos1_hbm.at[pl.ds(off, CC)], idx1_v)
        cp = pltpu.async_copy(cheap_hbm.at[pl.ds(off, CC)], acc_v, sem2)
        g0 = pltpu.async_copy(ys_hbm.at[idx0_v], r0_v, sem)
        g1 = pltpu.async_copy(ys_hbm.at[idx1_v], r1_v, sem)
        cp.wait()
        g0.wait()
        g1.wait()

        def _row(i, _):
            def _lane(j, _):
                sl = pl.ds(j * 16, 16)
                acc_v[i, sl] = acc_v[i, sl] + r0_v[i, sl] + r1_v[i, sl]
                return 0
            return lax.fori_loop(0, D // 16, _lane, 0)

        lax.fori_loop(0, CC, _row, 0)
        pltpu.sync_copy(acc_v, out_hbm.at[pl.ds(off, CC)])


def kernel(x, wg, W1, b1, W2, b2, cw, cconst):
    xf = x.reshape(T, D)
    cwf = jnp.concatenate([cw[0], cw[1]], axis=1)
    cheap, gates, idx = _router(xf, wg, cwf, cconst)

    pair_e = idx.reshape(-1)
    pair_g = gates.reshape(-1)
    pair_t = jnp.repeat(jnp.arange(T, dtype=jnp.int32), TOPK)
    is_ffn = pair_e < NF
    ec = jnp.where(is_ffn, pair_e, 0)
    onehot = (pair_e[:, None]
              == jnp.arange(NF, dtype=jnp.int32)[None, :]).astype(jnp.int32)
    csum = jnp.cumsum(onehot, axis=0)
    rank = jnp.take_along_axis(csum, ec[:, None], axis=1)[:, 0] - 1
    counts = csum[-1]
    padded = ((counts + B - 1) // B) * B
    po = jnp.concatenate(
        [jnp.zeros((1,), jnp.int32), jnp.cumsum(padded)]).astype(jnp.int32)
    dest = po[ec] + rank
    dest_s = jnp.where(is_ffn, dest, NPAD)
    tok_sorted = jnp.zeros((NPAD,), jnp.int32).at[dest_s].set(
        pair_t, mode="drop")
    gate_sorted = jnp.zeros((NPAD,), jnp.float32).at[dest_s].set(
        pair_g, mode="drop")
    pos = jnp.where(is_ffn, dest, ZROW).reshape(T, TOPK)
    n_active = (po[NF] // B).reshape(1).astype(jnp.int32)
    bstart = jnp.arange(NB, dtype=jnp.int32) * B
    block_expert = jnp.minimum(
        jnp.sum((bstart[:, None] >= po[None, 1:NF + 1]).astype(jnp.int32),
                axis=1),
        NF - 1).astype(jnp.int32)

    xs = _sc_gather(xf, tok_sorted)

    W1b = W1.astype(jnp.bfloat16)
    W2b = W2.astype(jnp.bfloat16)
    b1r = b1.reshape(NF, 1, F)
    b2r = b2.reshape(NF, 1, D)
    ys = _ffn(block_expert, n_active, xs, W1b, b1r, W2b, b2r,
              gate_sorted[:, None])

    pos0 = pos[:, 0] + 0
    pos1 = pos[:, 1] + 0
    out = _sc_combine(ys, cheap, pos0, pos1)
    return out.reshape(x.shape)
